# Initial kernel scaffold; baseline (speedup 1.0000x reference)
import jax
import jax.numpy as jnp
from jax.experimental import pallas as pl


def kernel(emb_table, edge_attrs, scale, edge_index):
    raise NotImplementedError("write your pallas kernel here")


# speedup vs baseline: 11.7151x; 11.7151x over previous
import functools
import jax
import jax.numpy as jnp
from jax import lax
from jax.experimental import pallas as pl
from jax.experimental.pallas import tpu as pltpu
from jax.experimental.pallas import tpu_sc as plsc

_N = 50000
_NP = 50048
_E = 800000
_D = 64
_H = _D // 2
_NS = 16
_EPS = _E // _NS
_K = 400
_CH = _EPS // _K
_RPS = _NP // _NS


def _agg_kernel(w_hbm, edge_hbm, zero_hbm, out_hbm,
                from_v, to_v, gidx_v, rows_v, acc, sem):
    c = lax.axis_index("c")
    s = lax.axis_index("s")

    r0 = s * _RPS
    pltpu.sync_copy(zero_hbm.at[pl.ds(r0, _RPS)], acc.at[pl.ds(r0, _RPS)])
    plsc.subcore_barrier()

    coff_v = jnp.full((16,), c * _NP, dtype=jnp.int32)
    base_e = s * _EPS

    def chunk_body(i, _):
        off = base_e + i * _K
        pltpu.sync_copy(edge_hbm.at[pl.ds(off, _K)], from_v)
        pltpu.sync_copy(edge_hbm.at[pl.ds(_E + off, _K)], to_v)

        def idx_body(j, _):
            v = from_v[pl.ds(j * 16, 16)] + coff_v
            gidx_v[pl.ds(j * 16, 16)] = v
            return 0

        lax.fori_loop(0, _K // 16, idx_body, 0)

        pltpu.async_copy(w_hbm.at[gidx_v], rows_v, sem).wait()
        pltpu.sync_copy(rows_v, acc.at[to_v], add=True)
        return 0

    lax.fori_loop(0, _CH, chunk_body, 0)
    plsc.subcore_barrier()

    pltpu.sync_copy(acc.at[pl.ds(r0, _RPS)],
                    out_hbm.at[pl.ds(c * _NP + r0, _RPS)])


@jax.jit
def _aggregate(w_stack, edge_flat, zeros_half):
    mesh = plsc.VectorSubcoreMesh(core_axis_name="c", subcore_axis_name="s")
    kern = functools.partial(
        pl.kernel,
        mesh=mesh,
        out_type=jax.ShapeDtypeStruct((2 * _NP, _H), jnp.float32),
        scratch_types=[
            pltpu.VMEM((_K,), jnp.int32),
            pltpu.VMEM((_K,), jnp.int32),
            pltpu.VMEM((_K,), jnp.int32),
            pltpu.VMEM((_K, _H), jnp.float32),
            pltpu.VMEM_SHARED((_NP, _H), jnp.float32),
            pltpu.SemaphoreType.DMA,
        ],
        compiler_params=pltpu.CompilerParams(use_tc_tiling_on_sc=False),
    )(_agg_kernel)
    return kern(w_stack, edge_flat, zeros_half)


def kernel(emb_table, edge_attrs, scale, edge_index):
    del edge_attrs, scale
    x0 = emb_table.astype(jnp.float32)
    zeros_half = jnp.zeros((_NP, _H), jnp.float32)

    edge_flat = edge_index.reshape(-1)

    ones_w = jnp.ones((2 * _NP, _H), jnp.float32)
    deg = _aggregate(ones_w, edge_flat, zeros_half)[:_N, 0]
    dis = jnp.where(deg > 0, lax.rsqrt(deg), 0.0)
    dis_p = jnp.concatenate([dis, jnp.zeros((_NP - _N,), jnp.float32)])
    dis_st = jnp.concatenate([dis_p, dis_p])[:, None]

    pad = jnp.zeros((_NP - _N, _H), jnp.float32)

    def stack(y):
        return jnp.concatenate([y[:_N, :_H], pad, y[:_N, _H:], pad], axis=0)

    w = stack(jnp.concatenate([dis[:, None] * x0], axis=1))
    acc_st = jnp.zeros((2 * _NP, _H), jnp.float32)
    for _ in range(3):
        z = _aggregate(w, edge_flat, zeros_half)
        x_st = dis_st * z
        acc_st = acc_st + x_st
        w = dis_st * x_st

    acc = jnp.concatenate([acc_st[:_N], acc_st[_NP:_NP + _N]], axis=1)
    return (x0 + acc) * jnp.float32(0.25)


# speedup vs baseline: 16.7781x; 1.4322x over previous
import functools
import jax
import jax.numpy as jnp
from jax import lax
from jax.experimental import pallas as pl
from jax.experimental.pallas import tpu as pltpu
from jax.experimental.pallas import tpu_sc as plsc

_N = 50000
_NP = 50048
_E = 800000
_D = 64
_H = _D // 2
_NS = 16
_EPS = _E // _NS
_K = 400
_CH = _EPS // _K
_RPS = _NP // _NS


def _zero_acc(zero_hbm, acc, s):
    r0 = s * _RPS
    pltpu.sync_copy(zero_hbm.at[pl.ds(r0, _RPS)], acc.at[pl.ds(r0, _RPS)])
    plsc.subcore_barrier()
    return r0


def _flush_acc(acc, out_hbm, c, r0):
    plsc.subcore_barrier()
    pltpu.sync_copy(acc.at[pl.ds(r0, _RPS)],
                    out_hbm.at[pl.ds(c * _NP + r0, _RPS)])


def _agg_kernel(w_hbm, edge_hbm, zero_hbm, out_hbm,
                from0, to0, gidx0, rows0,
                from1, to1, gidx1, rows1,
                acc, gsem0, gsem1):
    c = lax.axis_index("c")
    s = lax.axis_index("s")
    r0 = _zero_acc(zero_hbm, acc, s)

    coff_v = jnp.full((16,), c * _NP, dtype=jnp.int32)
    base_e = s * _EPS

    def load_idx(off, from_v, to_v, gidx_v):
        pltpu.sync_copy(edge_hbm.at[pl.ds(off, _K)], from_v)
        pltpu.sync_copy(edge_hbm.at[pl.ds(_E + off, _K)], to_v)

        def idx_body(j, _):
            gidx_v[pl.ds(j * 16, 16)] = from_v[pl.ds(j * 16, 16)] + coff_v
            return 0

        lax.fori_loop(0, _K // 16, idx_body, 0)

    load_idx(base_e, from0, to0, gidx0)
    pltpu.async_copy(w_hbm.at[gidx0], rows0, gsem0)

    def pair_body(t, _):
        load_idx(base_e + (2 * t + 1) * _K, from1, to1, gidx1)
        pltpu.make_async_copy(w_hbm.at[gidx0], rows0, gsem0).wait()
        pltpu.async_copy(w_hbm.at[gidx1], rows1, gsem1)
        pltpu.sync_copy(rows0, acc.at[to0], add=True)
        load_idx(base_e + (2 * t + 2) * _K, from0, to0, gidx0)
        pltpu.make_async_copy(w_hbm.at[gidx1], rows1, gsem1).wait()
        pltpu.async_copy(w_hbm.at[gidx0], rows0, gsem0)
        pltpu.sync_copy(rows1, acc.at[to1], add=True)
        return 0

    lax.fori_loop(0, (_CH - 1) // 2, pair_body, 0)

    pltpu.make_async_copy(w_hbm.at[gidx0], rows0, gsem0).wait()
    pltpu.sync_copy(rows0, acc.at[to0], add=True)

    _flush_acc(acc, out_hbm, c, r0)


def _deg_kernel(edge_hbm, zero_hbm, ones_hbm, out_hbm,
                to_v, ones_v, acc):
    c = lax.axis_index("c")
    s = lax.axis_index("s")
    r0 = _zero_acc(zero_hbm, acc, s)

    pltpu.sync_copy(ones_hbm, ones_v)
    base_e = s * _EPS

    def chunk_body(i, _):
        pltpu.sync_copy(edge_hbm.at[pl.ds(_E + base_e + i * _K, _K)], to_v)
        pltpu.sync_copy(ones_v, acc.at[to_v], add=True)
        return 0

    lax.fori_loop(0, _CH, chunk_body, 0)
    _flush_acc(acc, out_hbm, c, r0)


@jax.jit
def _aggregate(w_stack, edge_flat, zeros_half):
    mesh = plsc.VectorSubcoreMesh(core_axis_name="c", subcore_axis_name="s")
    kern = functools.partial(
        pl.kernel,
        mesh=mesh,
        out_type=jax.ShapeDtypeStruct((2 * _NP, _H), jnp.float32),
        scratch_types=[
            pltpu.VMEM((_K,), jnp.int32),
            pltpu.VMEM((_K,), jnp.int32),
            pltpu.VMEM((_K,), jnp.int32),
            pltpu.VMEM((_K, _H), jnp.float32),
            pltpu.VMEM((_K,), jnp.int32),
            pltpu.VMEM((_K,), jnp.int32),
            pltpu.VMEM((_K,), jnp.int32),
            pltpu.VMEM((_K, _H), jnp.float32),
            pltpu.VMEM_SHARED((_NP, _H), jnp.float32),
            pltpu.SemaphoreType.DMA,
            pltpu.SemaphoreType.DMA,
        ],
        compiler_params=pltpu.CompilerParams(use_tc_tiling_on_sc=False),
    )(_agg_kernel)
    return kern(w_stack, edge_flat, zeros_half)


@jax.jit
def _degrees(edge_flat, zeros_half, ones_chunk):
    mesh = plsc.VectorSubcoreMesh(core_axis_name="c", subcore_axis_name="s")
    kern = functools.partial(
        pl.kernel,
        mesh=mesh,
        out_type=jax.ShapeDtypeStruct((2 * _NP, _H), jnp.float32),
        scratch_types=[
            pltpu.VMEM((_K,), jnp.int32),
            pltpu.VMEM((_K, _H), jnp.float32),
            pltpu.VMEM_SHARED((_NP, _H), jnp.float32),
        ],
        compiler_params=pltpu.CompilerParams(use_tc_tiling_on_sc=False),
    )(_deg_kernel)
    return kern(edge_flat, zeros_half, ones_chunk)


def kernel(emb_table, edge_attrs, scale, edge_index):
    del edge_attrs, scale
    x0 = emb_table.astype(jnp.float32)
    zeros_half = jnp.zeros((_NP, _H), jnp.float32)
    edge_flat = edge_index.reshape(-1)

    ones_chunk = jnp.ones((_K, _H), jnp.float32)
    deg = _degrees(edge_flat, zeros_half, ones_chunk)[:_N, 0]
    dis = jnp.where(deg > 0, lax.rsqrt(deg), 0.0)
    dis_p = jnp.concatenate([dis, jnp.zeros((_NP - _N,), jnp.float32)])
    dis_st = jnp.concatenate([dis_p, dis_p])[:, None]

    pad = jnp.zeros((_NP - _N, _H), jnp.float32)

    def stack(y):
        return jnp.concatenate([y[:_N, :_H], pad, y[:_N, _H:], pad], axis=0)

    w = stack(dis[:, None] * x0)
    acc_st = jnp.zeros((2 * _NP, _H), jnp.float32)
    for _ in range(3):
        z = _aggregate(w, edge_flat, zeros_half)
        x_st = dis_st * z
        acc_st = acc_st + x_st
        w = dis_st * x_st

    acc = jnp.concatenate([acc_st[:_N], acc_st[_NP:_NP + _N]], axis=1)
    return (x0 + acc) * jnp.float32(0.25)


# speedup vs baseline: 18.9623x; 1.1302x over previous
import functools
import jax
import jax.numpy as jnp
from jax import lax
from jax.experimental import pallas as pl
from jax.experimental.pallas import tpu as pltpu
from jax.experimental.pallas import tpu_sc as plsc

_N = 50000
_NP = 50048
_E = 800000
_D = 64
_H = _D // 2
_NS = 16
_EPS = _E // _NS
_K = 400
_CH = _EPS // _K
_RPS = _NP // _NS


def _zero_acc(zero_hbm, acc, s):
    r0 = s * _RPS
    pltpu.sync_copy(zero_hbm.at[pl.ds(r0, _RPS)], acc.at[pl.ds(r0, _RPS)])
    plsc.subcore_barrier()
    return r0


def _flush_acc(acc, out_hbm, c, r0):
    plsc.subcore_barrier()
    pltpu.sync_copy(acc.at[pl.ds(r0, _RPS)],
                    out_hbm.at[pl.ds(c * _NP + r0, _RPS)])


def _agg_kernel(w_hbm, edge_hbm, zero_hbm, out_hbm,
                from0, to0, gidx0, rows0,
                from1, to1, gidx1, rows1,
                acc, gsem0, gsem1, isem0, isem1):
    c = lax.axis_index("c")
    s = lax.axis_index("s")
    r0 = _zero_acc(zero_hbm, acc, s)

    coff_v = jnp.full((16,), c * _NP, dtype=jnp.int32)
    base_e = s * _EPS
    pairs = (_CH - 1) // 2

    def issue_idx(off, from_v, to_v, isem):
        pltpu.async_copy(edge_hbm.at[pl.ds(off, _K)], from_v, isem)
        pltpu.async_copy(edge_hbm.at[pl.ds(_E + off, _K)], to_v, isem)

    def wait_idx(off, from_v, to_v, isem):
        pltpu.make_async_copy(edge_hbm.at[pl.ds(off, _K)], from_v, isem).wait()
        pltpu.make_async_copy(edge_hbm.at[pl.ds(_E + off, _K)], to_v,
                              isem).wait()

    def compute_gidx(from_v, gidx_v):
        def idx_body(j, _):
            gidx_v[pl.ds(j * 16, 16)] = from_v[pl.ds(j * 16, 16)] + coff_v
            return 0

        lax.fori_loop(0, _K // 16, idx_body, 0)

    pltpu.sync_copy(edge_hbm.at[pl.ds(base_e, _K)], from0)
    pltpu.sync_copy(edge_hbm.at[pl.ds(_E + base_e, _K)], to0)
    compute_gidx(from0, gidx0)
    pltpu.async_copy(w_hbm.at[gidx0], rows0, gsem0)
    issue_idx(base_e + _K, from1, to1, isem1)

    def pair_body(t, _):
        a = base_e + (2 * t + 1) * _K
        b = a + _K
        wait_idx(a, from1, to1, isem1)
        compute_gidx(from1, gidx1)
        pltpu.make_async_copy(w_hbm.at[gidx0], rows0, gsem0).wait()
        pltpu.async_copy(w_hbm.at[gidx1], rows1, gsem1)
        pltpu.sync_copy(rows0, acc.at[to0], add=True)
        issue_idx(b, from0, to0, isem0)
        wait_idx(b, from0, to0, isem0)
        compute_gidx(from0, gidx0)
        pltpu.make_async_copy(w_hbm.at[gidx1], rows1, gsem1).wait()
        pltpu.async_copy(w_hbm.at[gidx0], rows0, gsem0)
        pltpu.sync_copy(rows1, acc.at[to1], add=True)

        @pl.when(t < pairs - 1)
        def _():
            issue_idx(b + _K, from1, to1, isem1)

        return 0

    lax.fori_loop(0, pairs, pair_body, 0)

    pltpu.make_async_copy(w_hbm.at[gidx0], rows0, gsem0).wait()
    pltpu.sync_copy(rows0, acc.at[to0], add=True)

    _flush_acc(acc, out_hbm, c, r0)


def _deg_kernel(edge_hbm, zero_hbm, ones_hbm, out_hbm,
                to_v, ones_v, acc):
    c = lax.axis_index("c")
    s = lax.axis_index("s")
    r0 = _zero_acc(zero_hbm, acc, s)

    pltpu.sync_copy(ones_hbm, ones_v)
    base_e = s * _EPS

    def chunk_body(i, _):
        pltpu.sync_copy(edge_hbm.at[pl.ds(_E + base_e + i * _K, _K)], to_v)
        pltpu.sync_copy(ones_v, acc.at[to_v], add=True)
        return 0

    lax.fori_loop(0, _CH, chunk_body, 0)
    _flush_acc(acc, out_hbm, c, r0)


@jax.jit
def _aggregate(w_stack, edge_flat, zeros_half):
    mesh = plsc.VectorSubcoreMesh(core_axis_name="c", subcore_axis_name="s")
    kern = functools.partial(
        pl.kernel,
        mesh=mesh,
        out_type=jax.ShapeDtypeStruct((2 * _NP, _H), jnp.float32),
        scratch_types=[
            pltpu.VMEM((_K,), jnp.int32),
            pltpu.VMEM((_K,), jnp.int32),
            pltpu.VMEM((_K,), jnp.int32),
            pltpu.VMEM((_K, _H), jnp.float32),
            pltpu.VMEM((_K,), jnp.int32),
            pltpu.VMEM((_K,), jnp.int32),
            pltpu.VMEM((_K,), jnp.int32),
            pltpu.VMEM((_K, _H), jnp.float32),
            pltpu.VMEM_SHARED((_NP, _H), jnp.float32),
            pltpu.SemaphoreType.DMA,
            pltpu.SemaphoreType.DMA,
            pltpu.SemaphoreType.DMA,
            pltpu.SemaphoreType.DMA,
        ],
        compiler_params=pltpu.CompilerParams(use_tc_tiling_on_sc=False),
    )(_agg_kernel)
    return kern(w_stack, edge_flat, zeros_half)


@jax.jit
def _degrees(edge_flat, zeros_half, ones_chunk):
    mesh = plsc.VectorSubcoreMesh(core_axis_name="c", subcore_axis_name="s")
    kern = functools.partial(
        pl.kernel,
        mesh=mesh,
        out_type=jax.ShapeDtypeStruct((2 * _NP, _H), jnp.float32),
        scratch_types=[
            pltpu.VMEM((_K,), jnp.int32),
            pltpu.VMEM((_K, _H), jnp.float32),
            pltpu.VMEM_SHARED((_NP, _H), jnp.float32),
        ],
        compiler_params=pltpu.CompilerParams(use_tc_tiling_on_sc=False),
    )(_deg_kernel)
    return kern(edge_flat, zeros_half, ones_chunk)


def kernel(emb_table, edge_attrs, scale, edge_index):
    del edge_attrs, scale
    x0 = emb_table.astype(jnp.float32)
    zeros_half = jnp.zeros((_NP, _H), jnp.float32)
    edge_flat = edge_index.reshape(-1)

    ones_chunk = jnp.ones((_K, _H), jnp.float32)
    deg = _degrees(edge_flat, zeros_half, ones_chunk)[:_N, 0]
    dis = jnp.where(deg > 0, lax.rsqrt(deg), 0.0)
    dis_p = jnp.concatenate([dis, jnp.zeros((_NP - _N,), jnp.float32)])
    dis_st = jnp.concatenate([dis_p, dis_p])[:, None]

    pad = jnp.zeros((_NP - _N, _H), jnp.float32)

    def stack(y):
        return jnp.concatenate([y[:_N, :_H], pad, y[:_N, _H:], pad], axis=0)

    w = stack(dis[:, None] * x0)
    acc_st = jnp.zeros((2 * _NP, _H), jnp.float32)
    for _ in range(3):
        z = _aggregate(w, edge_flat, zeros_half)
        x_st = dis_st * z
        acc_st = acc_st + x_st
        w = dis_st * x_st

    acc = jnp.concatenate([acc_st[:_N], acc_st[_NP:_NP + _N]], axis=1)
    return (x0 + acc) * jnp.float32(0.25)
